# baseline (speedup 1.0000x reference)
import jax
import jax.numpy as jnp
from jax.experimental import pallas as pl

_K = 3
_EPS = 1e-10
_TILE = 1024


def _top3(s):
    half = s.shape[0] // 2
    lo = jnp.minimum(s[:half], s[half:])
    hi = jnp.maximum(s[:half], s[half:])

    hh = half // 2
    a0, b0 = lo[:hh], lo[hh:]
    a1, b1 = hi[:hh], hi[hh:]
    t0 = jnp.minimum(a0, b0)
    p = jnp.maximum(a0, b0)
    q = jnp.minimum(a1, b1)
    t1 = jnp.minimum(p, q)
    t2 = jnp.maximum(p, q)

    while t0.shape[0] > 1:
        h = t0.shape[0] // 2
        a0, b0 = t0[:h], t0[h:]
        a1, b1 = t1[:h], t1[h:]
        a2, b2 = t2[:h], t2[h:]
        t0 = jnp.minimum(a0, b0)
        p = jnp.maximum(a0, b0)
        q = jnp.minimum(a1, b1)
        t1 = jnp.minimum(p, q)
        r = jnp.maximum(a1, b1)
        u = jnp.minimum(a2, b2)
        t2 = jnp.minimum(jnp.minimum(jnp.maximum(p, q), r), u)
    return t0, t1, t2


def _fi_kernel(q_ref, k_ref, qq_ref, kk_ref, kf_ref, qf_ref, out_ref):
    q = q_ref[...]
    k = k_ref[...]
    n2 = k.shape[1]
    t = q.shape[1]

    qb = q.astype(jnp.bfloat16)
    kb = (-2.0 * k).astype(jnp.bfloat16)
    cross = jax.lax.dot_general(
        kb, qb, (((0,), (0,)), ((), ())),
        preferred_element_type=jnp.float32)
    qq = qq_ref[...]
    kk = kk_ref[...]
    d2 = (qq + cross) + kk

    m1, m2, m3 = _top3(d2)

    inv1 = 1.0 / jnp.maximum(m1, _EPS)
    inv2 = 1.0 / jnp.maximum(m2, _EPS)
    inv3 = 1.0 / jnp.maximum(m3, _EPS)
    norm = (inv1 + inv2) + inv3

    w = jnp.where(
        d2 == m1, inv1,
        jnp.where(d2 == m2, inv2,
                  jnp.where(d2 == m3, inv3, 0.0))).astype(jnp.bfloat16)

    kf = kf_ref[...]
    interp = jax.lax.dot_general(
        kf, w, (((1,), (0,)), ((), ())),
        preferred_element_type=jnp.float32)
    c = kf.shape[0]
    out_ref[0:c, :] = interp * (1.0 / norm)
    out_ref[c:, :] = qf_ref[...]


@jax.jit
def kernel(query_xyz, key_xyz, query_feature, key_feature):
    B, _, N1 = query_xyz.shape
    C2, N2 = key_feature.shape[1], key_feature.shape[2]
    C1 = query_feature.shape[1]

    qq = jnp.sum(query_xyz * query_xyz, axis=1)[:, None, :]
    kk = jnp.sum(key_xyz * key_xyz, axis=1)[:, :, None]
    kf16 = key_feature.astype(jnp.bfloat16)

    grid = (B, N1 // _TILE)
    return pl.pallas_call(
        _fi_kernel,
        grid=grid,
        in_specs=[
            pl.BlockSpec((None, 3, _TILE), lambda b, t: (b, 0, t)),
            pl.BlockSpec((None, 3, N2), lambda b, t: (b, 0, 0)),
            pl.BlockSpec((None, 1, _TILE), lambda b, t: (b, 0, t)),
            pl.BlockSpec((None, N2, 1), lambda b, t: (b, 0, 0)),
            pl.BlockSpec((None, C2, N2), lambda b, t: (b, 0, 0)),
            pl.BlockSpec((None, C1, _TILE), lambda b, t: (b, 0, t)),
        ],
        out_specs=pl.BlockSpec((None, C2 + C1, _TILE), lambda b, t: (b, 0, t)),
        out_shape=jax.ShapeDtypeStruct((B, C2 + C1, N1), jnp.float32),
    )(query_xyz, key_xyz, qq, kk, kf16, query_feature)


# speedup vs baseline: 1.0678x; 1.0678x over previous
import jax
import jax.numpy as jnp
from jax.experimental import pallas as pl

_K = 3
_EPS = 1e-10
_TILE = 1024


def _top3(s):
    half = s.shape[0] // 2
    lo = jnp.minimum(s[:half], s[half:])
    hi = jnp.maximum(s[:half], s[half:])

    hh = half // 2
    a0, b0 = lo[:hh], lo[hh:]
    a1, b1 = hi[:hh], hi[hh:]
    t0 = jnp.minimum(a0, b0)
    p = jnp.maximum(a0, b0)
    q = jnp.minimum(a1, b1)
    t1 = jnp.minimum(p, q)
    t2 = jnp.maximum(p, q)

    while t0.shape[0] > 1:
        h = t0.shape[0] // 2
        a0, b0 = t0[:h], t0[h:]
        a1, b1 = t1[:h], t1[h:]
        a2, b2 = t2[:h], t2[h:]
        t0 = jnp.minimum(a0, b0)
        p = jnp.maximum(a0, b0)
        q = jnp.minimum(a1, b1)
        t1 = jnp.minimum(p, q)
        r = jnp.maximum(a1, b1)
        u = jnp.minimum(a2, b2)
        t2 = jnp.minimum(jnp.minimum(jnp.maximum(p, q), r), u)
    return t0, t1, t2


def _fi_kernel(q_ref, k_ref, qq_ref, kk_ref, kf_ref, qf_ref, out_ref):
    q = q_ref[...]
    k = k_ref[...]
    n2 = k.shape[1]
    t = q.shape[1]

    qb = q.astype(jnp.bfloat16)
    kb = (-2.0 * k).astype(jnp.bfloat16)
    cross = jax.lax.dot_general(
        kb, qb, (((0,), (0,)), ((), ())),
        preferred_element_type=jnp.float32)
    qq = qq_ref[...]
    kk = kk_ref[...]
    d2 = (qq + cross) + kk

    m1, m2, m3 = _top3(d2)

    inv1 = 1.0 / jnp.maximum(m1, _EPS)
    inv2 = 1.0 / jnp.maximum(m2, _EPS)
    inv3 = 1.0 / jnp.maximum(m3, _EPS)
    norm = (inv1 + inv2) + inv3

    w = jnp.where(
        d2 == m1, inv1,
        jnp.where(d2 == m2, inv2,
                  jnp.where(d2 == m3, inv3, 0.0)))

    kf = kf_ref[...]
    interp = jax.lax.dot_general(
        kf, w, (((1,), (0,)), ((), ())),
        preferred_element_type=jnp.float32)
    c = kf.shape[0]
    out_ref[0:c, :] = interp * (1.0 / norm)
    out_ref[c:, :] = qf_ref[...]


@jax.jit
def kernel(query_xyz, key_xyz, query_feature, key_feature):
    B, _, N1 = query_xyz.shape
    C2, N2 = key_feature.shape[1], key_feature.shape[2]
    C1 = query_feature.shape[1]

    qq = jnp.sum(query_xyz * query_xyz, axis=1)[:, None, :]
    kk = jnp.sum(key_xyz * key_xyz, axis=1)[:, :, None]

    grid = (B, N1 // _TILE)
    return pl.pallas_call(
        _fi_kernel,
        grid=grid,
        in_specs=[
            pl.BlockSpec((None, 3, _TILE), lambda b, t: (b, 0, t)),
            pl.BlockSpec((None, 3, N2), lambda b, t: (b, 0, 0)),
            pl.BlockSpec((None, 1, _TILE), lambda b, t: (b, 0, t)),
            pl.BlockSpec((None, N2, 1), lambda b, t: (b, 0, 0)),
            pl.BlockSpec((None, C2, N2), lambda b, t: (b, 0, 0)),
            pl.BlockSpec((None, C1, _TILE), lambda b, t: (b, 0, t)),
        ],
        out_specs=pl.BlockSpec((None, C2 + C1, _TILE), lambda b, t: (b, 0, t)),
        out_shape=jax.ShapeDtypeStruct((B, C2 + C1, N1), jnp.float32),
    )(query_xyz, key_xyz, qq, kk, key_feature, query_feature)


# speedup vs baseline: 1.0694x; 1.0015x over previous
import jax
import jax.numpy as jnp
from jax.experimental import pallas as pl
from jax.experimental.pallas import tpu as pltpu

_K = 3
_EPS = 1e-10
_TILE = 1024


def _top3(s):
    half = s.shape[0] // 2
    lo = jnp.minimum(s[:half], s[half:])
    hi = jnp.maximum(s[:half], s[half:])

    hh = half // 2
    a0, b0 = lo[:hh], lo[hh:]
    a1, b1 = hi[:hh], hi[hh:]
    t0 = jnp.minimum(a0, b0)
    p = jnp.maximum(a0, b0)
    q = jnp.minimum(a1, b1)
    t1 = jnp.minimum(p, q)
    t2 = jnp.maximum(p, q)

    while t0.shape[0] > 1:
        h = t0.shape[0] // 2
        a0, b0 = t0[:h], t0[h:]
        a1, b1 = t1[:h], t1[h:]
        a2, b2 = t2[:h], t2[h:]
        t0 = jnp.minimum(a0, b0)
        p = jnp.maximum(a0, b0)
        q = jnp.minimum(a1, b1)
        t1 = jnp.minimum(p, q)
        r = jnp.maximum(a1, b1)
        u = jnp.minimum(a2, b2)
        t2 = jnp.minimum(jnp.minimum(jnp.maximum(p, q), r), u)
    return t0, t1, t2


def _fi_kernel(q_ref, k_ref, qq_ref, kk_ref, kf_ref, qf_ref, out_ref):
    q = q_ref[...]
    k = k_ref[...]
    n2 = k.shape[1]
    t = q.shape[1]

    qb = q.astype(jnp.bfloat16)
    kb = (-2.0 * k).astype(jnp.bfloat16)
    cross = jax.lax.dot_general(
        kb, qb, (((0,), (0,)), ((), ())),
        preferred_element_type=jnp.float32)
    qq = qq_ref[...]
    kk = kk_ref[...]
    d2 = (qq + cross) + kk

    m1, m2, m3 = _top3(d2)

    inv1 = 1.0 / jnp.maximum(m1, _EPS)
    inv2 = 1.0 / jnp.maximum(m2, _EPS)
    inv3 = 1.0 / jnp.maximum(m3, _EPS)
    norm = (inv1 + inv2) + inv3

    w = jnp.where(
        d2 == m1, inv1,
        jnp.where(d2 == m2, inv2,
                  jnp.where(d2 == m3, inv3, 0.0)))

    kf = kf_ref[...]
    interp = jax.lax.dot_general(
        kf, w, (((1,), (0,)), ((), ())),
        preferred_element_type=jnp.float32)
    c = kf.shape[0]
    out_ref[0:c, :] = interp * (1.0 / norm)
    out_ref[c:, :] = qf_ref[...]


@jax.jit
def kernel(query_xyz, key_xyz, query_feature, key_feature):
    B, _, N1 = query_xyz.shape
    C2, N2 = key_feature.shape[1], key_feature.shape[2]
    C1 = query_feature.shape[1]

    qq = jnp.sum(query_xyz * query_xyz, axis=1)[:, None, :]
    kk = jnp.sum(key_xyz * key_xyz, axis=1)[:, :, None]

    grid = (B, N1 // _TILE)
    return pl.pallas_call(
        _fi_kernel,
        grid=grid,
        in_specs=[
            pl.BlockSpec((None, 3, _TILE), lambda b, t: (b, 0, t)),
            pl.BlockSpec((None, 3, N2), lambda b, t: (b, 0, 0)),
            pl.BlockSpec((None, 1, _TILE), lambda b, t: (b, 0, t)),
            pl.BlockSpec((None, N2, 1), lambda b, t: (b, 0, 0)),
            pl.BlockSpec((None, C2, N2), lambda b, t: (b, 0, 0)),
            pl.BlockSpec((None, C1, _TILE), lambda b, t: (b, 0, t)),
        ],
        out_specs=pl.BlockSpec((None, C2 + C1, _TILE), lambda b, t: (b, 0, t)),
        out_shape=jax.ShapeDtypeStruct((B, C2 + C1, N1), jnp.float32),
        compiler_params=pltpu.CompilerParams(
            dimension_semantics=("parallel", "parallel")),
    )(query_xyz, key_xyz, qq, kk, key_feature, query_feature)


# speedup vs baseline: 1.0712x; 1.0017x over previous
import jax
import jax.numpy as jnp
from jax.experimental import pallas as pl
from jax.experimental.pallas import tpu as pltpu

_K = 3
_EPS = 1e-10
_TILE = 1024


def _top3(s):
    half = s.shape[0] // 2
    lo = jnp.minimum(s[:half], s[half:])
    hi = jnp.maximum(s[:half], s[half:])

    hh = half // 2
    a0, b0 = lo[:hh], lo[hh:]
    a1, b1 = hi[:hh], hi[hh:]
    t0 = jnp.minimum(a0, b0)
    p = jnp.maximum(a0, b0)
    q = jnp.minimum(a1, b1)
    t1 = jnp.minimum(p, q)
    t2 = jnp.maximum(p, q)

    while t0.shape[0] > 1:
        h = t0.shape[0] // 2
        a0, b0 = t0[:h], t0[h:]
        a1, b1 = t1[:h], t1[h:]
        a2, b2 = t2[:h], t2[h:]
        t0 = jnp.minimum(a0, b0)
        p = jnp.maximum(a0, b0)
        q = jnp.minimum(a1, b1)
        t1 = jnp.minimum(p, q)
        r = jnp.maximum(a1, b1)
        u = jnp.minimum(a2, b2)
        t2 = jnp.minimum(jnp.minimum(jnp.maximum(p, q), r), u)
    return t0, t1, t2


def _fi_kernel(q_ref, k_ref, qq_ref, kk_ref, kf_ref, qf_ref, out_ref):
    q = q_ref[...]
    k = k_ref[...]
    n2 = k.shape[1]
    t = q.shape[1]

    qb = q.astype(jnp.bfloat16)
    kb = (-2.0 * k).astype(jnp.bfloat16)
    cross = jax.lax.dot_general(
        kb, qb, (((0,), (0,)), ((), ())),
        preferred_element_type=jnp.float32)
    qq = qq_ref[...]
    kk = kk_ref[...]
    d2 = (qq + cross) + kk

    m1, m2, m3 = _top3(d2)

    inv1 = 1.0 / jnp.maximum(m1, _EPS)
    inv2 = 1.0 / jnp.maximum(m2, _EPS)
    inv3 = 1.0 / jnp.maximum(m3, _EPS)
    rnorm = 1.0 / ((inv1 + inv2) + inv3)
    inv1 = inv1 * rnorm
    inv2 = inv2 * rnorm
    inv3 = inv3 * rnorm

    w = jnp.where(
        d2 == m1, inv1,
        jnp.where(d2 == m2, inv2,
                  jnp.where(d2 == m3, inv3, 0.0)))

    kf = kf_ref[...]
    interp = jax.lax.dot_general(
        kf, w, (((1,), (0,)), ((), ())),
        preferred_element_type=jnp.float32)
    c = kf.shape[0]
    out_ref[0:c, :] = interp
    out_ref[c:, :] = qf_ref[...]


@jax.jit
def kernel(query_xyz, key_xyz, query_feature, key_feature):
    B, _, N1 = query_xyz.shape
    C2, N2 = key_feature.shape[1], key_feature.shape[2]
    C1 = query_feature.shape[1]

    qq = jnp.sum(query_xyz * query_xyz, axis=1)[:, None, :]
    kk = jnp.sum(key_xyz * key_xyz, axis=1)[:, :, None]

    grid = (B, N1 // _TILE)
    return pl.pallas_call(
        _fi_kernel,
        grid=grid,
        in_specs=[
            pl.BlockSpec((None, 3, _TILE), lambda b, t: (b, 0, t)),
            pl.BlockSpec((None, 3, N2), lambda b, t: (b, 0, 0)),
            pl.BlockSpec((None, 1, _TILE), lambda b, t: (b, 0, t)),
            pl.BlockSpec((None, N2, 1), lambda b, t: (b, 0, 0)),
            pl.BlockSpec((None, C2, N2), lambda b, t: (b, 0, 0)),
            pl.BlockSpec((None, C1, _TILE), lambda b, t: (b, 0, t)),
        ],
        out_specs=pl.BlockSpec((None, C2 + C1, _TILE), lambda b, t: (b, 0, t)),
        out_shape=jax.ShapeDtypeStruct((B, C2 + C1, N1), jnp.float32),
        compiler_params=pltpu.CompilerParams(
            dimension_semantics=("parallel", "parallel")),
    )(query_xyz, key_xyz, qq, kk, key_feature, query_feature)
